# baseline (speedup 1.0000x reference)
import jax
import jax.numpy as jnp
from jax import lax
from jax.experimental import pallas as pl
from jax.experimental.pallas import tpu as pltpu
from jax.experimental.pallas import tpu_sc as plsc

N = 2048
D = 1024
E = 8
T = 256
NSLOT = 24
NROWS = NSLOT * T
SEG_SZ = N + 16
NPAIR_PAD = 2 * N + 32
CHUNKS = N // 16


def _router_body(x_ref, gwt_ref, idx_ref, w_ref):
    x = x_ref[...]
    lg = jnp.dot(x, gwt_ref[...], preferred_element_type=jnp.float32)
    m = jnp.max(lg, axis=-1, keepdims=True)
    ex = jnp.exp(lg - m)
    s = ex / jnp.sum(ex, axis=-1, keepdims=True)
    m1 = s[:, 0:1]
    i1 = jnp.zeros_like(m1, dtype=jnp.int32)
    m2 = jnp.full_like(m1, -1.0)
    i2 = jnp.full_like(i1, -1)
    for e in range(1, E):
        v = s[:, e : e + 1]
        gt1 = v > m1
        gt2 = v > m2
        m2n = jnp.where(gt1, m1, jnp.where(gt2, v, m2))
        i2n = jnp.where(gt1, i1, jnp.where(gt2, e, i2))
        m1 = jnp.where(gt1, v, m1)
        i1 = jnp.where(gt1, e, i1)
        m2, i2 = m2n, i2n
    idx_ref[:, 0:1] = i1
    idx_ref[:, 1:2] = i2
    idx_ref[:, 2:8] = jnp.zeros((i1.shape[0], 6), jnp.int32)
    w_ref[:, 0:1] = m1
    w_ref[:, 1:2] = m2
    w_ref[:, 2:8] = jnp.zeros((i1.shape[0], 6), jnp.float32)


def _meta_body(idx0h, idx1h, sorted_tok, pairdst, slots, idxbuf0, idxbuf1,
               seg, segp, zbuf, slotsv):
    sid = lax.axis_index("s")
    iota = lax.iota(jnp.int32, 16)
    zeros16 = jnp.zeros((16,), jnp.int32)
    dummy = jnp.full((16,), 2 * N, jnp.int32) + sid

    @pl.when(sid >= E)
    def _zero_fill():
        share = NROWS // E
        base = (sid - E) * share

        def zf(j, _):
            zbuf[pl.ds(pl.multiple_of(j * 16, 16), 16)] = zeros16
            return 0

        lax.fori_loop(0, 16, zf, 0)
        for j in range(share // 256):
            pltpu.sync_copy(
                zbuf, sorted_tok.at[pl.ds(pl.multiple_of(base + j * 256, 256), 256)])

        def zf2(j, _):
            zbuf[pl.ds(pl.multiple_of(j * 16, 16), 16)] = dummy
            return 0

        lax.fori_loop(0, 16, zf2, 0)
        for j in range(share // 256):
            pltpu.sync_copy(
                zbuf, pairdst.at[pl.ds(pl.multiple_of(base + j * 256, 256), 256)])

    pltpu.sync_copy(idx0h, idxbuf0)
    pltpu.sync_copy(idx1h, idxbuf1)

    def c_body(c, accs):
        i0 = idxbuf0[pl.ds(c * 16, 16)]
        i1 = idxbuf1[pl.ds(c * 16, 16)]
        return tuple(
            accs[e]
            + jnp.where(i0 == e, 1, 0)
            + jnp.where(i1 == e, 1, 0)
            for e in range(E)
        )

    accs = lax.fori_loop(0, CHUNKS, c_body, tuple(zeros16 for _ in range(E)))
    cnts = [jnp.sum(accs[e]) for e in range(E)]
    nts = [(cnts[e] + (T - 1)) // T for e in range(E)]
    seg_base = jnp.int32(0)
    ntiles_mine = jnp.int32(0)
    total_slots = jnp.int32(0)
    for e in range(E):
        seg_base = seg_base + jnp.where(sid > e, nts[e], 0) * T
        ntiles_mine = ntiles_mine + jnp.where(sid == e, nts[e], 0)
        total_slots = total_slots + nts[e]

    def i_body(j, _):
        seg[pl.ds(pl.multiple_of(j * 16, 16), 16)] = zeros16
        segp[pl.ds(pl.multiple_of(j * 16, 16), 16)] = dummy
        return 0

    lax.fori_loop(0, SEG_SZ // 16, i_body, 0)

    def p_body(c, off):
        tok = c * 16 + iota
        i0 = idxbuf0[pl.ds(c * 16, 16)]
        i1 = idxbuf1[pl.ds(c * 16, 16)]
        m0 = i0 == sid
        c0 = jnp.sum(jnp.where(m0, 1, 0))
        plsc.store_compressed(seg.at[pl.ds(off, 16)], tok, mask=m0)
        plsc.store_compressed(segp.at[pl.ds(off, 16)], tok, mask=m0)
        m1 = i1 == sid
        c1 = jnp.sum(jnp.where(m1, 1, 0))
        plsc.store_compressed(seg.at[pl.ds(off + c0, 16)], tok, mask=m1)
        plsc.store_compressed(segp.at[pl.ds(off + c0, 16)], tok + N, mask=m1)
        return off + c0 + c1

    lax.fori_loop(0, CHUNKS, p_body, jnp.int32(0))

    @pl.when(sid == 0)
    def _slots():
        for c in range(2):
            p = c * 16 + iota
            v = jnp.zeros((16,), jnp.int32)
            sb = jnp.int32(0)
            for e in range(E):
                v = v + jnp.where((p >= sb) & (p < sb + nts[e]), e, 0)
                sb = sb + nts[e]
            v = v + jnp.where(p == NSLOT, total_slots, 0)
            slotsv[pl.ds(c * 16, 16)] = v

    plsc.subcore_barrier()

    def d_body(j, _):
        src = pl.ds(pl.multiple_of(j * T, T), T)
        dst = pl.ds(pl.multiple_of(seg_base + j * T, T), T)
        pltpu.sync_copy(seg.at[src], sorted_tok.at[dst])
        pltpu.sync_copy(segp.at[src], pairdst.at[dst])
        return 0

    lax.fori_loop(0, ntiles_mine, d_body, 0)

    @pl.when(sid == 0)
    def _wslots():
        pltpu.sync_copy(slotsv, slots)


def _xgather_body(x_hbm, st_hbm, xs_hbm, idxall, rows0, rows1, sg0, sg1, sw0, sw1):
    wid = lax.axis_index("s") * 2 + lax.axis_index("c")
    base = pl.multiple_of(wid * 192, 64)
    b0 = pl.ds(base, 96)
    b1 = pl.ds(pl.multiple_of(base + 96, 32), 96)
    pltpu.sync_copy(st_hbm.at[pl.ds(base, 192)], idxall)
    g0 = pltpu.async_copy(x_hbm.at[idxall.at[pl.ds(0, 96)]], rows0, sg0)
    g1 = pltpu.async_copy(x_hbm.at[idxall.at[pl.ds(96, 96)]], rows1, sg1)
    g0.wait()
    w0 = pltpu.async_copy(rows0, xs_hbm.at[b0], sw0)
    g1.wait()
    w1 = pltpu.async_copy(rows1, xs_hbm.at[b1], sw1)
    w0.wait()
    w1.wait()


def _oscatter_body(ob_hbm, pd_hbm, yp_hbm, idxall, rows0, rows1, sg0, sg1, sw0, sw1):
    wid = lax.axis_index("s") * 2 + lax.axis_index("c")
    base = pl.multiple_of(wid * 192, 64)
    b0 = pl.ds(base, 96)
    b1 = pl.ds(pl.multiple_of(base + 96, 32), 96)
    pltpu.sync_copy(pd_hbm.at[pl.ds(base, 192)], idxall)
    r0 = pltpu.async_copy(ob_hbm.at[b0], rows0, sg0)
    r1 = pltpu.async_copy(ob_hbm.at[b1], rows1, sg1)
    r0.wait()
    s0 = pltpu.async_copy(rows0, yp_hbm.at[idxall.at[pl.ds(0, 96)]], sw0)
    r1.wait()
    s1 = pltpu.async_copy(rows1, yp_hbm.at[idxall.at[pl.ds(96, 96)]], sw1)
    s0.wait()
    s1.wait()


def _expert_body(slots_ref, x_ref, wg_ref, wu_ref, wd_ref, out_ref):
    t = pl.program_id(0)

    @pl.when(t < slots_ref[NSLOT])
    def _go():
        x = x_ref[...]
        xg = jnp.dot(x, wg_ref[0], preferred_element_type=jnp.float32)
        xu = jnp.dot(x, wu_ref[0], preferred_element_type=jnp.float32)
        h = (xg * jax.nn.sigmoid(xg) * xu).astype(jnp.bfloat16)
        out_ref[...] = jnp.dot(
            h, wd_ref[0], preferred_element_type=jnp.float32).astype(jnp.bfloat16)


def _combine_body(x_ref, b0_ref, b1_ref, w_ref, swg_ref, swu_ref, swd_ref, y_ref):
    x = x_ref[...]
    xg = jnp.dot(x, swg_ref[...], preferred_element_type=jnp.float32)
    xu = jnp.dot(x, swu_ref[...], preferred_element_type=jnp.float32)
    h = (xg * jax.nn.sigmoid(xg) * xu).astype(jnp.bfloat16)
    acc = jnp.dot(h, swd_ref[...], preferred_element_type=jnp.float32)
    acc = (acc + b0_ref[...].astype(jnp.float32) * w_ref[:, 0:1]
           + b1_ref[...].astype(jnp.float32) * w_ref[:, 1:2])
    y_ref[...] = acc


def kernel(hidden_states, gate_weight, w_gate, w_up, w_down, sw_gate, sw_up, sw_down):
    Bsz, S, _ = hidden_states.shape
    F = w_gate.shape[2]
    FS = sw_gate.shape[1]
    nt = N // T

    x = hidden_states.reshape(N, D)
    x16 = x.astype(jnp.bfloat16)
    gwt = gate_weight.T
    wg16 = w_gate.astype(jnp.bfloat16)
    wu16 = w_up.astype(jnp.bfloat16)
    wd16 = w_down.astype(jnp.bfloat16)
    swg16 = sw_gate.astype(jnp.bfloat16)
    swu16 = sw_up.astype(jnp.bfloat16)
    swd16 = sw_down.astype(jnp.bfloat16)

    idx2d, w2d = pl.pallas_call(
        _router_body,
        grid=(nt,),
        in_specs=[
            pl.BlockSpec((T, D), lambda t: (t, 0)),
            pl.BlockSpec((D, E), lambda t: (0, 0)),
        ],
        out_specs=[
            pl.BlockSpec((T, E), lambda t: (t, 0)),
            pl.BlockSpec((T, E), lambda t: (t, 0)),
        ],
        out_shape=[
            jax.ShapeDtypeStruct((N, E), jnp.int32),
            jax.ShapeDtypeStruct((N, E), jnp.float32),
        ],
    )(x, gwt)

    sorted_tok, pairdst, slots = pl.kernel(
        _meta_body,
        out_type=[
            jax.ShapeDtypeStruct((NROWS,), jnp.int32),
            jax.ShapeDtypeStruct((NROWS,), jnp.int32),
            jax.ShapeDtypeStruct((32,), jnp.int32),
        ],
        mesh=plsc.VectorSubcoreMesh(
            core_axis_name="c", subcore_axis_name="s", num_cores=1),
        compiler_params=pltpu.CompilerParams(needs_layout_passes=False),
        scratch_types=[
            pltpu.VMEM((N,), jnp.int32),
            pltpu.VMEM((N,), jnp.int32),
            pltpu.VMEM((SEG_SZ,), jnp.int32),
            pltpu.VMEM((SEG_SZ,), jnp.int32),
            pltpu.VMEM((256,), jnp.int32),
            pltpu.VMEM((32,), jnp.int32),
        ],
    )(idx2d[:, 0], idx2d[:, 1])

    x16i = lax.bitcast_convert_type(x16.reshape(N, D // 2, 2), jnp.int32)
    x_sorted_i = pl.kernel(
        _xgather_body,
        out_type=jax.ShapeDtypeStruct((NROWS, D // 2), jnp.int32),
        mesh=plsc.VectorSubcoreMesh(core_axis_name="c", subcore_axis_name="s"),
        compiler_params=pltpu.CompilerParams(needs_layout_passes=False),
        scratch_types=[
            pltpu.VMEM((192,), jnp.int32),
            pltpu.VMEM((96, D // 2), jnp.int32),
            pltpu.VMEM((96, D // 2), jnp.int32),
            pltpu.SemaphoreType.DMA,
            pltpu.SemaphoreType.DMA,
            pltpu.SemaphoreType.DMA,
            pltpu.SemaphoreType.DMA,
        ],
    )(x16i, sorted_tok)
    x_sorted = lax.bitcast_convert_type(
        x_sorted_i, jnp.bfloat16).reshape(NROWS, D)

    out_buf = pl.pallas_call(
        _expert_body,
        grid_spec=pltpu.PrefetchScalarGridSpec(
            num_scalar_prefetch=1,
            grid=(NSLOT,),
            in_specs=[
                pl.BlockSpec((T, D), lambda t, m: (t, 0)),
                pl.BlockSpec((1, D, F), lambda t, m: (m[t], 0, 0)),
                pl.BlockSpec((1, D, F), lambda t, m: (m[t], 0, 0)),
                pl.BlockSpec((1, F, D), lambda t, m: (m[t], 0, 0)),
            ],
            out_specs=pl.BlockSpec((T, D), lambda t, m: (t, 0)),
        ),
        out_shape=jax.ShapeDtypeStruct((NROWS, D), jnp.bfloat16),
    )(slots, x_sorted, wg16, wu16, wd16)

    out_i = lax.bitcast_convert_type(
        out_buf.reshape(NROWS, D // 2, 2), jnp.int32)
    ypairs_i = pl.kernel(
        _oscatter_body,
        out_type=jax.ShapeDtypeStruct((NPAIR_PAD, D // 2), jnp.int32),
        mesh=plsc.VectorSubcoreMesh(core_axis_name="c", subcore_axis_name="s"),
        compiler_params=pltpu.CompilerParams(needs_layout_passes=False),
        scratch_types=[
            pltpu.VMEM((192,), jnp.int32),
            pltpu.VMEM((96, D // 2), jnp.int32),
            pltpu.VMEM((96, D // 2), jnp.int32),
            pltpu.SemaphoreType.DMA,
            pltpu.SemaphoreType.DMA,
            pltpu.SemaphoreType.DMA,
            pltpu.SemaphoreType.DMA,
        ],
    )(out_i, pairdst)
    ypairs = lax.bitcast_convert_type(
        ypairs_i, jnp.bfloat16).reshape(NPAIR_PAD, D)

    y = pl.pallas_call(
        _combine_body,
        grid=(nt,),
        in_specs=[
            pl.BlockSpec((T, D), lambda t: (t, 0)),
            pl.BlockSpec((T, D), lambda t: (t, 0)),
            pl.BlockSpec((T, D), lambda t: (t + N // T, 0)),
            pl.BlockSpec((T, E), lambda t: (t, 0)),
            pl.BlockSpec((D, FS), lambda t: (0, 0)),
            pl.BlockSpec((D, FS), lambda t: (0, 0)),
            pl.BlockSpec((FS, D), lambda t: (0, 0)),
        ],
        out_specs=pl.BlockSpec((T, D), lambda t: (t, 0)),
        out_shape=jax.ShapeDtypeStruct((N, D), jnp.float32),
    )(x16, ypairs, ypairs, w2d, swg16, swu16, swd16)

    return y.reshape(Bsz, S, D)


# speedup vs baseline: 1.5819x; 1.5819x over previous
import jax
import jax.numpy as jnp
from jax import lax
from jax.experimental import pallas as pl
from jax.experimental.pallas import tpu as pltpu
from jax.experimental.pallas import tpu_sc as plsc

N = 2048
D = 1024
E = 8
T = 256
NSLOT = 24
NROWS = NSLOT * T
SEG_SZ = N + 16
NPAIR_PAD = 2 * N + 32
CHUNKS = N // 16


def _router_body(x_ref, gwt_ref, idx_ref, w_ref):
    x = x_ref[...]
    lg = jnp.dot(x, gwt_ref[...], preferred_element_type=jnp.float32)
    m = jnp.max(lg, axis=-1, keepdims=True)
    ex = jnp.exp(lg - m)
    s = ex / jnp.sum(ex, axis=-1, keepdims=True)
    m1 = s[:, 0:1]
    i1 = jnp.zeros_like(m1, dtype=jnp.int32)
    m2 = jnp.full_like(m1, -1.0)
    i2 = jnp.full_like(i1, -1)
    for e in range(1, E):
        v = s[:, e : e + 1]
        gt1 = v > m1
        gt2 = v > m2
        m2n = jnp.where(gt1, m1, jnp.where(gt2, v, m2))
        i2n = jnp.where(gt1, i1, jnp.where(gt2, e, i2))
        m1 = jnp.where(gt1, v, m1)
        i1 = jnp.where(gt1, e, i1)
        m2, i2 = m2n, i2n
    idx_ref[:, 0:1] = i1
    idx_ref[:, 1:2] = i2
    idx_ref[:, 2:8] = jnp.zeros((i1.shape[0], 6), jnp.int32)
    w_ref[:, 0:1] = m1
    w_ref[:, 1:2] = m2
    w_ref[:, 2:8] = jnp.zeros((i1.shape[0], 6), jnp.float32)


def _meta_body(idx0h, idx1h, sorted_tok, pairdst, slots, idxbuf0, idxbuf1,
               seg, segp, zbuf, slotsv):
    sid = lax.axis_index("s")
    iota = lax.iota(jnp.int32, 16)
    zeros16 = jnp.zeros((16,), jnp.int32)
    dummy = jnp.full((16,), 2 * N, jnp.int32) + sid

    @pl.when(sid >= E)
    def _zero_fill():
        share = NROWS // E
        base = (sid - E) * share

        def zf(j, _):
            zbuf[pl.ds(pl.multiple_of(j * 16, 16), 16)] = zeros16
            return 0

        lax.fori_loop(0, 16, zf, 0)
        for j in range(share // 256):
            pltpu.sync_copy(
                zbuf, sorted_tok.at[pl.ds(pl.multiple_of(base + j * 256, 256), 256)])

        def zf2(j, _):
            zbuf[pl.ds(pl.multiple_of(j * 16, 16), 16)] = dummy
            return 0

        lax.fori_loop(0, 16, zf2, 0)
        for j in range(share // 256):
            pltpu.sync_copy(
                zbuf, pairdst.at[pl.ds(pl.multiple_of(base + j * 256, 256), 256)])

    pltpu.sync_copy(idx0h, idxbuf0)
    pltpu.sync_copy(idx1h, idxbuf1)

    def c_body(c, accs):
        i0 = idxbuf0[pl.ds(c * 16, 16)]
        i1 = idxbuf1[pl.ds(c * 16, 16)]
        return tuple(
            accs[e]
            + jnp.where(i0 == e, 1, 0)
            + jnp.where(i1 == e, 1, 0)
            for e in range(E)
        )

    accs = lax.fori_loop(0, CHUNKS, c_body, tuple(zeros16 for _ in range(E)))
    cnts = [jnp.sum(accs[e]) for e in range(E)]
    nts = [(cnts[e] + (T - 1)) // T for e in range(E)]
    seg_base = jnp.int32(0)
    ntiles_mine = jnp.int32(0)
    total_slots = jnp.int32(0)
    for e in range(E):
        seg_base = seg_base + jnp.where(sid > e, nts[e], 0) * T
        ntiles_mine = ntiles_mine + jnp.where(sid == e, nts[e], 0)
        total_slots = total_slots + nts[e]

    def i_body(j, _):
        seg[pl.ds(pl.multiple_of(j * 16, 16), 16)] = zeros16
        segp[pl.ds(pl.multiple_of(j * 16, 16), 16)] = dummy
        return 0

    lax.fori_loop(0, SEG_SZ // 16, i_body, 0)

    def p_body(c, off):
        tok = c * 16 + iota
        i0 = idxbuf0[pl.ds(c * 16, 16)]
        i1 = idxbuf1[pl.ds(c * 16, 16)]
        m0 = i0 == sid
        c0 = jnp.sum(jnp.where(m0, 1, 0))
        plsc.store_compressed(seg.at[pl.ds(off, 16)], tok, mask=m0)
        plsc.store_compressed(segp.at[pl.ds(off, 16)], tok, mask=m0)
        m1 = i1 == sid
        c1 = jnp.sum(jnp.where(m1, 1, 0))
        plsc.store_compressed(seg.at[pl.ds(off + c0, 16)], tok, mask=m1)
        plsc.store_compressed(segp.at[pl.ds(off + c0, 16)], tok + N, mask=m1)
        return off + c0 + c1

    lax.fori_loop(0, CHUNKS, p_body, jnp.int32(0))

    @pl.when(sid == 0)
    def _slots():
        for c in range(2):
            p = c * 16 + iota
            v = jnp.zeros((16,), jnp.int32)
            sb = jnp.int32(0)
            for e in range(E):
                v = v + jnp.where((p >= sb) & (p < sb + nts[e]), e, 0)
                sb = sb + nts[e]
            v = v + jnp.where(p == NSLOT, total_slots, 0)
            slotsv[pl.ds(c * 16, 16)] = v

    plsc.subcore_barrier()

    def d_body(j, _):
        src = pl.ds(pl.multiple_of(j * T, T), T)
        dst = pl.ds(pl.multiple_of(seg_base + j * T, T), T)
        pltpu.sync_copy(seg.at[src], sorted_tok.at[dst])
        pltpu.sync_copy(segp.at[src], pairdst.at[dst])
        return 0

    lax.fori_loop(0, ntiles_mine, d_body, 0)

    @pl.when(sid == 0)
    def _wslots():
        pltpu.sync_copy(slotsv, slots)


def _xgather_body(x_hbm, st_hbm, xs_hbm, idxall, rows0, rows1, sg0, sg1, sw0, sw1):
    wid = lax.axis_index("s") * 2 + lax.axis_index("c")
    base = pl.multiple_of(wid * 192, 64)
    pltpu.sync_copy(st_hbm.at[pl.ds(base, 192)], idxall)
    rows = (rows0, rows1)
    sg = (sg0, sg1)
    sw = (sw0, sw1)
    blk = [pl.ds(pl.multiple_of(base + j * 48, 16), 48) for j in range(4)]
    idx = [idxall.at[pl.ds(j * 48, 48)] for j in range(4)]
    g = [None] * 4
    w = [None] * 4
    g[0] = pltpu.async_copy(x_hbm.at[idx[0]], rows[0], sg[0])
    g[1] = pltpu.async_copy(x_hbm.at[idx[1]], rows[1], sg[1])
    for j in range(4):
        g[j].wait()
        w[j] = pltpu.async_copy(rows[j % 2], xs_hbm.at[blk[j]], sw[j % 2])
        if j + 2 < 4:
            w[j].wait()
            g[j + 2] = pltpu.async_copy(x_hbm.at[idx[j + 2]], rows[j % 2], sg[j % 2])
    w[2].wait()
    w[3].wait()


def _oscatter_body(ob_hbm, pd_hbm, yp_hbm, idxall, rows0, rows1, sg0, sg1, sw0, sw1):
    wid = lax.axis_index("s") * 2 + lax.axis_index("c")
    base = pl.multiple_of(wid * 192, 64)
    pltpu.sync_copy(pd_hbm.at[pl.ds(base, 192)], idxall)
    rows = (rows0, rows1)
    sg = (sg0, sg1)
    sw = (sw0, sw1)
    blk = [pl.ds(pl.multiple_of(base + j * 48, 16), 48) for j in range(4)]
    idx = [idxall.at[pl.ds(j * 48, 48)] for j in range(4)]
    g = [None] * 4
    w = [None] * 4
    g[0] = pltpu.async_copy(ob_hbm.at[blk[0]], rows[0], sg[0])
    g[1] = pltpu.async_copy(ob_hbm.at[blk[1]], rows[1], sg[1])
    for j in range(4):
        g[j].wait()
        w[j] = pltpu.async_copy(rows[j % 2], yp_hbm.at[idx[j]], sw[j % 2])
        if j + 2 < 4:
            w[j].wait()
            g[j + 2] = pltpu.async_copy(ob_hbm.at[blk[j + 2]], rows[j % 2], sg[j % 2])
    w[2].wait()
    w[3].wait()


def _expert_body(slots_ref, x_ref, wg_ref, wu_ref, wd_ref, out_ref):
    t = pl.program_id(0)

    @pl.when(t < slots_ref[NSLOT])
    def _go():
        x = x_ref[...].astype(jnp.bfloat16)
        xg = jnp.dot(x, wg_ref[0], preferred_element_type=jnp.float32)
        xu = jnp.dot(x, wu_ref[0], preferred_element_type=jnp.float32)
        h = (xg * jax.nn.sigmoid(xg) * xu).astype(jnp.bfloat16)
        out_ref[...] = jnp.dot(h, wd_ref[0], preferred_element_type=jnp.float32)


def _combine_body(x_ref, b0_ref, b1_ref, w_ref, swg_ref, swu_ref, swd_ref, y_ref):
    x = x_ref[...]
    xg = jnp.dot(x, swg_ref[...], preferred_element_type=jnp.float32)
    xu = jnp.dot(x, swu_ref[...], preferred_element_type=jnp.float32)
    h = (xg * jax.nn.sigmoid(xg) * xu).astype(jnp.bfloat16)
    acc = jnp.dot(h, swd_ref[...], preferred_element_type=jnp.float32)
    acc = acc + b0_ref[...] * w_ref[:, 0:1] + b1_ref[...] * w_ref[:, 1:2]
    y_ref[...] = acc


def kernel(hidden_states, gate_weight, w_gate, w_up, w_down, sw_gate, sw_up, sw_down):
    Bsz, S, _ = hidden_states.shape
    F = w_gate.shape[2]
    FS = sw_gate.shape[1]
    nt = N // T

    x = hidden_states.reshape(N, D)
    x16 = x.astype(jnp.bfloat16)
    gwt = gate_weight.T
    wg16 = w_gate.astype(jnp.bfloat16)
    wu16 = w_up.astype(jnp.bfloat16)
    wd16 = w_down.astype(jnp.bfloat16)
    swg16 = sw_gate.astype(jnp.bfloat16)
    swu16 = sw_up.astype(jnp.bfloat16)
    swd16 = sw_down.astype(jnp.bfloat16)

    idx2d, w2d = pl.pallas_call(
        _router_body,
        grid=(nt,),
        in_specs=[
            pl.BlockSpec((T, D), lambda t: (t, 0)),
            pl.BlockSpec((D, E), lambda t: (0, 0)),
        ],
        out_specs=[
            pl.BlockSpec((T, E), lambda t: (t, 0)),
            pl.BlockSpec((T, E), lambda t: (t, 0)),
        ],
        out_shape=[
            jax.ShapeDtypeStruct((N, E), jnp.int32),
            jax.ShapeDtypeStruct((N, E), jnp.float32),
        ],
    )(x, gwt)

    sorted_tok, pairdst, slots = pl.kernel(
        _meta_body,
        out_type=[
            jax.ShapeDtypeStruct((NROWS,), jnp.int32),
            jax.ShapeDtypeStruct((NROWS,), jnp.int32),
            jax.ShapeDtypeStruct((32,), jnp.int32),
        ],
        mesh=plsc.VectorSubcoreMesh(
            core_axis_name="c", subcore_axis_name="s", num_cores=1),
        compiler_params=pltpu.CompilerParams(needs_layout_passes=False),
        scratch_types=[
            pltpu.VMEM((N,), jnp.int32),
            pltpu.VMEM((N,), jnp.int32),
            pltpu.VMEM((SEG_SZ,), jnp.int32),
            pltpu.VMEM((SEG_SZ,), jnp.int32),
            pltpu.VMEM((256,), jnp.int32),
            pltpu.VMEM((32,), jnp.int32),
        ],
    )(idx2d[:, 0], idx2d[:, 1])

    x_sorted = pl.kernel(
        _xgather_body,
        out_type=jax.ShapeDtypeStruct((NROWS, D), jnp.float32),
        mesh=plsc.VectorSubcoreMesh(core_axis_name="c", subcore_axis_name="s"),
        compiler_params=pltpu.CompilerParams(needs_layout_passes=False),
        scratch_types=[
            pltpu.VMEM((192,), jnp.int32),
            pltpu.VMEM((48, D), jnp.float32),
            pltpu.VMEM((48, D), jnp.float32),
            pltpu.SemaphoreType.DMA,
            pltpu.SemaphoreType.DMA,
            pltpu.SemaphoreType.DMA,
            pltpu.SemaphoreType.DMA,
        ],
    )(x, sorted_tok)

    out_buf = pl.pallas_call(
        _expert_body,
        grid_spec=pltpu.PrefetchScalarGridSpec(
            num_scalar_prefetch=1,
            grid=(NSLOT,),
            in_specs=[
                pl.BlockSpec((T, D), lambda t, m: (t, 0)),
                pl.BlockSpec((1, D, F), lambda t, m: (m[t], 0, 0)),
                pl.BlockSpec((1, D, F), lambda t, m: (m[t], 0, 0)),
                pl.BlockSpec((1, F, D), lambda t, m: (m[t], 0, 0)),
            ],
            out_specs=pl.BlockSpec((T, D), lambda t, m: (t, 0)),
        ),
        out_shape=jax.ShapeDtypeStruct((NROWS, D), jnp.float32),
    )(slots, x_sorted, wg16, wu16, wd16)

    ypairs = pl.kernel(
        _oscatter_body,
        out_type=jax.ShapeDtypeStruct((NPAIR_PAD, D), jnp.float32),
        mesh=plsc.VectorSubcoreMesh(core_axis_name="c", subcore_axis_name="s"),
        compiler_params=pltpu.CompilerParams(needs_layout_passes=False),
        scratch_types=[
            pltpu.VMEM((192,), jnp.int32),
            pltpu.VMEM((48, D), jnp.float32),
            pltpu.VMEM((48, D), jnp.float32),
            pltpu.SemaphoreType.DMA,
            pltpu.SemaphoreType.DMA,
            pltpu.SemaphoreType.DMA,
            pltpu.SemaphoreType.DMA,
        ],
    )(out_buf, pairdst)

    y = pl.pallas_call(
        _combine_body,
        grid=(nt,),
        in_specs=[
            pl.BlockSpec((T, D), lambda t: (t, 0)),
            pl.BlockSpec((T, D), lambda t: (t, 0)),
            pl.BlockSpec((T, D), lambda t: (t + N // T, 0)),
            pl.BlockSpec((T, E), lambda t: (t, 0)),
            pl.BlockSpec((D, FS), lambda t: (0, 0)),
            pl.BlockSpec((D, FS), lambda t: (0, 0)),
            pl.BlockSpec((FS, D), lambda t: (0, 0)),
        ],
        out_specs=pl.BlockSpec((T, D), lambda t: (t, 0)),
        out_shape=jax.ShapeDtypeStruct((N, D), jnp.float32),
    )(x16, ypairs, ypairs, w2d, swg16, swu16, swd16)

    return y.reshape(Bsz, S, D)


# speedup vs baseline: 1.7449x; 1.1031x over previous
import jax
import jax.numpy as jnp
from jax import lax
from jax.experimental import pallas as pl
from jax.experimental.pallas import tpu as pltpu
from jax.experimental.pallas import tpu_sc as plsc

N = 2048
D = 1024
E = 8
T = 128
NSLOT = 40
NROWS = NSLOT * T
SEG_SZ = N + 16
NPAIR_PAD = 2 * N + 32
CHUNKS = N // 16


def _router_body(x_ref, gwt_ref, idx_ref, w_ref):
    x = x_ref[...]
    lg = jnp.dot(x, gwt_ref[...], preferred_element_type=jnp.float32)
    m = jnp.max(lg, axis=-1, keepdims=True)
    ex = jnp.exp(lg - m)
    s = ex / jnp.sum(ex, axis=-1, keepdims=True)
    m1 = s[:, 0:1]
    i1 = jnp.zeros_like(m1, dtype=jnp.int32)
    m2 = jnp.full_like(m1, -1.0)
    i2 = jnp.full_like(i1, -1)
    for e in range(1, E):
        v = s[:, e : e + 1]
        gt1 = v > m1
        gt2 = v > m2
        m2n = jnp.where(gt1, m1, jnp.where(gt2, v, m2))
        i2n = jnp.where(gt1, i1, jnp.where(gt2, e, i2))
        m1 = jnp.where(gt1, v, m1)
        i1 = jnp.where(gt1, e, i1)
        m2, i2 = m2n, i2n
    idx_ref[:, 0:1] = i1
    idx_ref[:, 1:2] = i2
    idx_ref[:, 2:8] = jnp.zeros((i1.shape[0], 6), jnp.int32)
    w_ref[:, 0:1] = m1
    w_ref[:, 1:2] = m2
    w_ref[:, 2:8] = jnp.zeros((i1.shape[0], 6), jnp.float32)


def _meta_body(idx0h, idx1h, sorted_tok, pairdst, slots, idxbuf0, idxbuf1,
               seg, segp, zbuf, slotsv):
    sid = lax.axis_index("s")
    iota = lax.iota(jnp.int32, 16)
    zeros16 = jnp.zeros((16,), jnp.int32)
    dummy = jnp.full((16,), 2 * N, jnp.int32) + sid

    @pl.when(sid >= E)
    def _zero_fill():
        share = NROWS // E
        base = (sid - E) * share

        def zf(j, _):
            zbuf[pl.ds(pl.multiple_of(j * 16, 16), 16)] = zeros16
            return 0

        lax.fori_loop(0, 8, zf, 0)
        for j in range(share // 128):
            pltpu.sync_copy(
                zbuf.at[pl.ds(0, 128)],
                sorted_tok.at[pl.ds(pl.multiple_of(base + j * 128, 128), 128)])

        def zf2(j, _):
            zbuf[pl.ds(pl.multiple_of(j * 16, 16), 16)] = dummy
            return 0

        lax.fori_loop(0, 8, zf2, 0)
        for j in range(share // 128):
            pltpu.sync_copy(
                zbuf.at[pl.ds(0, 128)],
                pairdst.at[pl.ds(pl.multiple_of(base + j * 128, 128), 128)])

    pltpu.sync_copy(idx0h, idxbuf0)
    pltpu.sync_copy(idx1h, idxbuf1)

    def c_body(c, accs):
        i0 = idxbuf0[pl.ds(c * 16, 16)]
        i1 = idxbuf1[pl.ds(c * 16, 16)]
        return tuple(
            accs[e]
            + jnp.where(i0 == e, 1, 0)
            + jnp.where(i1 == e, 1, 0)
            for e in range(E)
        )

    accs = lax.fori_loop(0, CHUNKS, c_body, tuple(zeros16 for _ in range(E)))
    cnts = [jnp.sum(accs[e]) for e in range(E)]
    nts = [(cnts[e] + (T - 1)) // T for e in range(E)]
    seg_base = jnp.int32(0)
    ntiles_mine = jnp.int32(0)
    total_slots = jnp.int32(0)
    for e in range(E):
        seg_base = seg_base + jnp.where(sid > e, nts[e], 0) * T
        ntiles_mine = ntiles_mine + jnp.where(sid == e, nts[e], 0)
        total_slots = total_slots + nts[e]

    def i_body(j, _):
        seg[pl.ds(pl.multiple_of(j * 16, 16), 16)] = zeros16
        segp[pl.ds(pl.multiple_of(j * 16, 16), 16)] = dummy
        return 0

    lax.fori_loop(0, SEG_SZ // 16, i_body, 0)

    def p_body(c, off):
        tok = c * 16 + iota
        i0 = idxbuf0[pl.ds(c * 16, 16)]
        i1 = idxbuf1[pl.ds(c * 16, 16)]
        m0 = i0 == sid
        c0 = jnp.sum(jnp.where(m0, 1, 0))
        plsc.store_compressed(seg.at[pl.ds(off, 16)], tok, mask=m0)
        plsc.store_compressed(segp.at[pl.ds(off, 16)], tok, mask=m0)
        m1 = i1 == sid
        c1 = jnp.sum(jnp.where(m1, 1, 0))
        plsc.store_compressed(seg.at[pl.ds(off + c0, 16)], tok, mask=m1)
        plsc.store_compressed(segp.at[pl.ds(off + c0, 16)], tok + N, mask=m1)
        return off + c0 + c1

    lax.fori_loop(0, CHUNKS, p_body, jnp.int32(0))

    @pl.when(sid == 0)
    def _slots():
        for c in range(3):
            p = c * 16 + iota
            v = jnp.zeros((16,), jnp.int32)
            sb = jnp.int32(0)
            for e in range(E):
                v = v + jnp.where((p >= sb) & (p < sb + nts[e]), e, 0)
                sb = sb + nts[e]
            v = v + jnp.where(p == NSLOT, total_slots, 0)
            slotsv[pl.ds(c * 16, 16)] = v

    plsc.subcore_barrier()

    def d_body(j, _):
        src = pl.ds(pl.multiple_of(j * T, T), T)
        dst = pl.ds(pl.multiple_of(seg_base + j * T, T), T)
        pltpu.sync_copy(seg.at[src], sorted_tok.at[dst])
        pltpu.sync_copy(segp.at[src], pairdst.at[dst])
        return 0

    lax.fori_loop(0, ntiles_mine, d_body, 0)

    @pl.when(sid == 0)
    def _wslots():
        pltpu.sync_copy(slotsv, slots)


def _xgather_body(x_hbm, st_hbm, xs_hbm, idxall, rows0, sg0, sw0):
    wid = lax.axis_index("s") * 2 + lax.axis_index("c")
    per = NROWS // 32
    base = pl.multiple_of(wid * per, 32)
    pltpu.sync_copy(st_hbm.at[pl.ds(base, per)], idxall)
    pltpu.async_copy(x_hbm.at[idxall.at[pl.ds(0, 96)]], rows0, sg0).wait()
    w0 = pltpu.async_copy(rows0, xs_hbm.at[pl.ds(base, 96)], sw0)
    w0.wait()
    pltpu.async_copy(
        x_hbm.at[idxall.at[pl.ds(96, 64)]], rows0.at[pl.ds(0, 64)], sg0).wait()
    pltpu.async_copy(
        rows0.at[pl.ds(0, 64)],
        xs_hbm.at[pl.ds(pl.multiple_of(base + 96, 32), 64)], sw0).wait()


def _oscatter_body(ob_hbm, pd_hbm, yp_hbm, idxall, rows0, sg0, sw0):
    wid = lax.axis_index("s") * 2 + lax.axis_index("c")
    per = NROWS // 32
    base = pl.multiple_of(wid * per, 32)
    pltpu.sync_copy(pd_hbm.at[pl.ds(base, per)], idxall)
    pltpu.async_copy(ob_hbm.at[pl.ds(base, 96)], rows0, sg0).wait()
    s0 = pltpu.async_copy(rows0, yp_hbm.at[idxall.at[pl.ds(0, 96)]], sw0)
    s0.wait()
    pltpu.async_copy(
        ob_hbm.at[pl.ds(pl.multiple_of(base + 96, 32), 64)],
        rows0.at[pl.ds(0, 64)], sg0).wait()
    pltpu.async_copy(
        rows0.at[pl.ds(0, 64)], yp_hbm.at[idxall.at[pl.ds(96, 64)]], sw0).wait()


def _expert_body(slots_ref, x_ref, wg_ref, wu_ref, wd_ref, out_ref):
    t = pl.program_id(0)

    @pl.when(t < slots_ref[NSLOT])
    def _go():
        x = x_ref[...].astype(jnp.bfloat16)
        xg = jnp.dot(x, wg_ref[0], preferred_element_type=jnp.float32)
        xu = jnp.dot(x, wu_ref[0], preferred_element_type=jnp.float32)
        h = (xg * jax.nn.sigmoid(xg) * xu).astype(jnp.bfloat16)
        out_ref[...] = jnp.dot(h, wd_ref[0], preferred_element_type=jnp.float32)


def _shared_body(x_ref, swg_ref, swu_ref, swd_ref, ysh_ref):
    x = x_ref[...]
    xg = jnp.dot(x, swg_ref[...], preferred_element_type=jnp.float32)
    xu = jnp.dot(x, swu_ref[...], preferred_element_type=jnp.float32)
    h = (xg * jax.nn.sigmoid(xg) * xu).astype(jnp.bfloat16)
    ysh_ref[...] = jnp.dot(h, swd_ref[...], preferred_element_type=jnp.float32)


def _combine_body(ysh_ref, b0_ref, b1_ref, w_ref, y_ref):
    y_ref[...] = (ysh_ref[...] + b0_ref[...] * w_ref[:, 0:1]
                  + b1_ref[...] * w_ref[:, 1:2])


def kernel(hidden_states, gate_weight, w_gate, w_up, w_down, sw_gate, sw_up, sw_down):
    Bsz, S, _ = hidden_states.shape
    F = w_gate.shape[2]
    FS = sw_gate.shape[1]
    nt = N // T

    x = hidden_states.reshape(N, D)
    x16 = x.astype(jnp.bfloat16)
    gwt = gate_weight.T
    wg16 = w_gate.astype(jnp.bfloat16)
    wu16 = w_up.astype(jnp.bfloat16)
    wd16 = w_down.astype(jnp.bfloat16)
    swg16 = sw_gate.astype(jnp.bfloat16)
    swu16 = sw_up.astype(jnp.bfloat16)
    swd16 = sw_down.astype(jnp.bfloat16)

    idx2d, w2d = pl.pallas_call(
        _router_body,
        grid=(nt,),
        in_specs=[
            pl.BlockSpec((T, D), lambda t: (t, 0)),
            pl.BlockSpec((D, E), lambda t: (0, 0)),
        ],
        out_specs=[
            pl.BlockSpec((T, E), lambda t: (t, 0)),
            pl.BlockSpec((T, E), lambda t: (t, 0)),
        ],
        out_shape=[
            jax.ShapeDtypeStruct((N, E), jnp.int32),
            jax.ShapeDtypeStruct((N, E), jnp.float32),
        ],
    )(x, gwt)

    CT = 256
    ysh = pl.pallas_call(
        _shared_body,
        grid=(N // CT,),
        in_specs=[
            pl.BlockSpec((CT, D), lambda t: (t, 0)),
            pl.BlockSpec((D, FS), lambda t: (0, 0)),
            pl.BlockSpec((D, FS), lambda t: (0, 0)),
            pl.BlockSpec((FS, D), lambda t: (0, 0)),
        ],
        out_specs=pl.BlockSpec((CT, D), lambda t: (t, 0)),
        out_shape=jax.ShapeDtypeStruct((N, D), jnp.float32),
    )(x16, swg16, swu16, swd16)

    sorted_tok, pairdst, slots = pl.kernel(
        _meta_body,
        out_type=[
            jax.ShapeDtypeStruct((NROWS,), jnp.int32),
            jax.ShapeDtypeStruct((NROWS,), jnp.int32),
            jax.ShapeDtypeStruct((48,), jnp.int32),
        ],
        mesh=plsc.VectorSubcoreMesh(
            core_axis_name="c", subcore_axis_name="s", num_cores=1),
        compiler_params=pltpu.CompilerParams(needs_layout_passes=False),
        scratch_types=[
            pltpu.VMEM((N,), jnp.int32),
            pltpu.VMEM((N,), jnp.int32),
            pltpu.VMEM((SEG_SZ,), jnp.int32),
            pltpu.VMEM((SEG_SZ,), jnp.int32),
            pltpu.VMEM((128,), jnp.int32),
            pltpu.VMEM((48,), jnp.int32),
        ],
    )(idx2d[:, 0], idx2d[:, 1])

    x_sorted = pl.kernel(
        _xgather_body,
        out_type=jax.ShapeDtypeStruct((NROWS, D), jnp.float32),
        mesh=plsc.VectorSubcoreMesh(core_axis_name="c", subcore_axis_name="s"),
        compiler_params=pltpu.CompilerParams(needs_layout_passes=False),
        scratch_types=[
            pltpu.VMEM((160,), jnp.int32),
            pltpu.VMEM((96, D), jnp.float32),
            pltpu.SemaphoreType.DMA,
            pltpu.SemaphoreType.DMA,
        ],
    )(x, sorted_tok)

    out_buf = pl.pallas_call(
        _expert_body,
        grid_spec=pltpu.PrefetchScalarGridSpec(
            num_scalar_prefetch=1,
            grid=(NSLOT,),
            in_specs=[
                pl.BlockSpec((T, D), lambda t, m: (t, 0)),
                pl.BlockSpec((1, D, F), lambda t, m: (m[t], 0, 0)),
                pl.BlockSpec((1, D, F), lambda t, m: (m[t], 0, 0)),
                pl.BlockSpec((1, F, D), lambda t, m: (m[t], 0, 0)),
            ],
            out_specs=pl.BlockSpec((T, D), lambda t, m: (t, 0)),
        ),
        out_shape=jax.ShapeDtypeStruct((NROWS, D), jnp.float32),
    )(slots, x_sorted, wg16, wu16, wd16)

    ypairs = pl.kernel(
        _oscatter_body,
        out_type=jax.ShapeDtypeStruct((NPAIR_PAD, D), jnp.float32),
        mesh=plsc.VectorSubcoreMesh(core_axis_name="c", subcore_axis_name="s"),
        compiler_params=pltpu.CompilerParams(needs_layout_passes=False),
        scratch_types=[
            pltpu.VMEM((160,), jnp.int32),
            pltpu.VMEM((96, D), jnp.float32),
            pltpu.SemaphoreType.DMA,
            pltpu.SemaphoreType.DMA,
        ],
    )(out_buf, pairdst)

    y = pl.pallas_call(
        _combine_body,
        grid=(N // CT,),
        in_specs=[
            pl.BlockSpec((CT, D), lambda t: (t, 0)),
            pl.BlockSpec((CT, D), lambda t: (t, 0)),
            pl.BlockSpec((CT, D), lambda t: (t + N // CT, 0)),
            pl.BlockSpec((CT, E), lambda t: (t, 0)),
        ],
        out_specs=pl.BlockSpec((CT, D), lambda t: (t, 0)),
        out_shape=jax.ShapeDtypeStruct((N, D), jnp.float32),
    )(ysh, ypairs, ypairs, w2d)

    return y.reshape(Bsz, S, D)


# speedup vs baseline: 1.9336x; 1.1081x over previous
import jax
import jax.numpy as jnp
from jax import lax
from jax.experimental import pallas as pl
from jax.experimental.pallas import tpu as pltpu
from jax.experimental.pallas import tpu_sc as plsc

N = 2048
D = 1024
E = 8
T = 128
NSLOT = 40
NROWS = NSLOT * T
SEG_SZ = N + 16
NPAIR_PAD = 2 * N + 32
CHUNKS = N // 16


def _router_body(x_ref, gwt_ref, swg_ref, swu_ref, swd_ref, idx_ref, w_ref, ysh_ref):
    x = x_ref[...]
    x16b = x.astype(jnp.bfloat16)
    xg = jnp.dot(x16b, swg_ref[...], preferred_element_type=jnp.float32)
    xu = jnp.dot(x16b, swu_ref[...], preferred_element_type=jnp.float32)
    hsh = (xg * jax.nn.sigmoid(xg) * xu).astype(jnp.bfloat16)
    ysh_ref[...] = jnp.dot(hsh, swd_ref[...], preferred_element_type=jnp.float32)
    lg = jnp.dot(x, gwt_ref[...], preferred_element_type=jnp.float32)
    m = jnp.max(lg, axis=-1, keepdims=True)
    ex = jnp.exp(lg - m)
    s = ex / jnp.sum(ex, axis=-1, keepdims=True)
    m1 = s[:, 0:1]
    i1 = jnp.zeros_like(m1, dtype=jnp.int32)
    m2 = jnp.full_like(m1, -1.0)
    i2 = jnp.full_like(i1, -1)
    for e in range(1, E):
        v = s[:, e : e + 1]
        gt1 = v > m1
        gt2 = v > m2
        m2n = jnp.where(gt1, m1, jnp.where(gt2, v, m2))
        i2n = jnp.where(gt1, i1, jnp.where(gt2, e, i2))
        m1 = jnp.where(gt1, v, m1)
        i1 = jnp.where(gt1, e, i1)
        m2, i2 = m2n, i2n
    idx_ref[:, 0:1] = i1
    idx_ref[:, 1:2] = i2
    idx_ref[:, 2:8] = jnp.zeros((i1.shape[0], 6), jnp.int32)
    w_ref[:, 0:1] = m1
    w_ref[:, 1:2] = m2
    w_ref[:, 2:8] = jnp.zeros((i1.shape[0], 6), jnp.float32)


def _meta_body(idx0h, idx1h, sorted_tok, pairdst, slots, idxbuf0, idxbuf1,
               seg, segp, zbuf, slotsv):
    sid = lax.axis_index("s")
    iota = lax.iota(jnp.int32, 16)
    zeros16 = jnp.zeros((16,), jnp.int32)
    dummy = jnp.full((16,), 2 * N, jnp.int32) + sid

    @pl.when(sid >= E)
    def _zero_fill():
        share = NROWS // E
        base = (sid - E) * share

        def zf(j, _):
            zbuf[pl.ds(pl.multiple_of(j * 16, 16), 16)] = zeros16
            return 0

        lax.fori_loop(0, 8, zf, 0)
        for j in range(share // 128):
            pltpu.sync_copy(
                zbuf.at[pl.ds(0, 128)],
                sorted_tok.at[pl.ds(pl.multiple_of(base + j * 128, 128), 128)])

        def zf2(j, _):
            zbuf[pl.ds(pl.multiple_of(j * 16, 16), 16)] = dummy
            return 0

        lax.fori_loop(0, 8, zf2, 0)
        for j in range(share // 128):
            pltpu.sync_copy(
                zbuf.at[pl.ds(0, 128)],
                pairdst.at[pl.ds(pl.multiple_of(base + j * 128, 128), 128)])

    pltpu.sync_copy(idx0h, idxbuf0)
    pltpu.sync_copy(idx1h, idxbuf1)

    def c_body(c, accs):
        i0 = idxbuf0[pl.ds(c * 16, 16)]
        i1 = idxbuf1[pl.ds(c * 16, 16)]
        return tuple(
            accs[e]
            + jnp.where(i0 == e, 1, 0)
            + jnp.where(i1 == e, 1, 0)
            for e in range(E)
        )

    accs = lax.fori_loop(0, CHUNKS, c_body, tuple(zeros16 for _ in range(E)))
    cnts = [jnp.sum(accs[e]) for e in range(E)]
    nts = [(cnts[e] + (T - 1)) // T for e in range(E)]
    seg_base = jnp.int32(0)
    ntiles_mine = jnp.int32(0)
    total_slots = jnp.int32(0)
    for e in range(E):
        seg_base = seg_base + jnp.where(sid > e, nts[e], 0) * T
        ntiles_mine = ntiles_mine + jnp.where(sid == e, nts[e], 0)
        total_slots = total_slots + nts[e]

    def i_body(j, _):
        seg[pl.ds(pl.multiple_of(j * 16, 16), 16)] = zeros16
        segp[pl.ds(pl.multiple_of(j * 16, 16), 16)] = dummy
        return 0

    lax.fori_loop(0, SEG_SZ // 16, i_body, 0)

    def p_body(c, off):
        tok = c * 16 + iota
        i0 = idxbuf0[pl.ds(c * 16, 16)]
        i1 = idxbuf1[pl.ds(c * 16, 16)]
        m0 = i0 == sid
        c0 = jnp.sum(jnp.where(m0, 1, 0))
        plsc.store_compressed(seg.at[pl.ds(off, 16)], tok, mask=m0)
        plsc.store_compressed(segp.at[pl.ds(off, 16)], tok, mask=m0)
        m1 = i1 == sid
        c1 = jnp.sum(jnp.where(m1, 1, 0))
        plsc.store_compressed(seg.at[pl.ds(off + c0, 16)], tok, mask=m1)
        plsc.store_compressed(segp.at[pl.ds(off + c0, 16)], tok + N, mask=m1)
        return off + c0 + c1

    lax.fori_loop(0, CHUNKS, p_body, jnp.int32(0))

    @pl.when(sid == 0)
    def _slots():
        for c in range(3):
            p = c * 16 + iota
            v = jnp.zeros((16,), jnp.int32)
            sb = jnp.int32(0)
            for e in range(E):
                v = v + jnp.where((p >= sb) & (p < sb + nts[e]), e, 0)
                sb = sb + nts[e]
            v = v + jnp.where(p == NSLOT, total_slots, 0)
            slotsv[pl.ds(c * 16, 16)] = v

    plsc.subcore_barrier()

    def d_body(j, _):
        src = pl.ds(pl.multiple_of(j * T, T), T)
        dst = pl.ds(pl.multiple_of(seg_base + j * T, T), T)
        pltpu.sync_copy(seg.at[src], sorted_tok.at[dst])
        pltpu.sync_copy(segp.at[src], pairdst.at[dst])
        return 0

    lax.fori_loop(0, ntiles_mine, d_body, 0)

    @pl.when(sid == 0)
    def _wslots():
        pltpu.sync_copy(slotsv, slots)


def _xgather_body(x_hbm, st_hbm, sl_hbm, xs_hbm, idxall, slv, rows0, sg0, sw0):
    wid = lax.axis_index("s") * 2 + lax.axis_index("c")
    per = NROWS // 32
    base = pl.multiple_of(wid * per, 32)
    pltpu.sync_copy(sl_hbm.at[pl.ds(32, 16)], slv)
    iota = lax.iota(jnp.int32, 16)
    limit = jnp.sum(jnp.where(iota == NSLOT - 32, slv[pl.ds(0, 16)], 0)) * T
    pltpu.sync_copy(st_hbm.at[pl.ds(base, per)], idxall)

    @pl.when(base < limit)
    def _c0():
        pltpu.async_copy(x_hbm.at[idxall.at[pl.ds(0, 96)]], rows0, sg0).wait()
        pltpu.async_copy(rows0, xs_hbm.at[pl.ds(base, 96)], sw0).wait()

    @pl.when(base + 96 < limit)
    def _c1():
        pltpu.async_copy(
            x_hbm.at[idxall.at[pl.ds(96, 64)]], rows0.at[pl.ds(0, 64)], sg0).wait()
        pltpu.async_copy(
            rows0.at[pl.ds(0, 64)],
            xs_hbm.at[pl.ds(pl.multiple_of(base + 96, 32), 64)], sw0).wait()


def _oscatter_body(ob_hbm, pd_hbm, sl_hbm, yp_hbm, idxall, slv, rows0, sg0, sw0):
    wid = lax.axis_index("s") * 2 + lax.axis_index("c")
    per = NROWS // 32
    base = pl.multiple_of(wid * per, 32)
    pltpu.sync_copy(sl_hbm.at[pl.ds(32, 16)], slv)
    iota = lax.iota(jnp.int32, 16)
    limit = jnp.sum(jnp.where(iota == NSLOT - 32, slv[pl.ds(0, 16)], 0)) * T
    pltpu.sync_copy(pd_hbm.at[pl.ds(base, per)], idxall)

    @pl.when(base < limit)
    def _c0():
        pltpu.async_copy(ob_hbm.at[pl.ds(base, 96)], rows0, sg0).wait()
        pltpu.async_copy(rows0, yp_hbm.at[idxall.at[pl.ds(0, 96)]], sw0).wait()

    @pl.when(base + 96 < limit)
    def _c1():
        pltpu.async_copy(
            ob_hbm.at[pl.ds(pl.multiple_of(base + 96, 32), 64)],
            rows0.at[pl.ds(0, 64)], sg0).wait()
        pltpu.async_copy(
            rows0.at[pl.ds(0, 64)], yp_hbm.at[idxall.at[pl.ds(96, 64)]], sw0).wait()


def _expert_body(slots_ref, x_ref, wg_ref, wu_ref, wd_ref, out_ref):
    t = pl.program_id(0)

    @pl.when(t < slots_ref[NSLOT])
    def _go():
        x = x_ref[...].astype(jnp.bfloat16)
        xg = jnp.dot(x, wg_ref[0], preferred_element_type=jnp.float32)
        xu = jnp.dot(x, wu_ref[0], preferred_element_type=jnp.float32)
        h = (xg * jax.nn.sigmoid(xg) * xu).astype(jnp.bfloat16)
        out_ref[...] = jnp.dot(h, wd_ref[0], preferred_element_type=jnp.float32)


def _combine_body(ysh_ref, b0_ref, b1_ref, w_ref, y_ref):
    y_ref[...] = (ysh_ref[...] + b0_ref[...] * w_ref[:, 0:1]
                  + b1_ref[...] * w_ref[:, 1:2])


def kernel(hidden_states, gate_weight, w_gate, w_up, w_down, sw_gate, sw_up, sw_down):
    Bsz, S, _ = hidden_states.shape
    F = w_gate.shape[2]
    FS = sw_gate.shape[1]
    nt = N // T

    x = hidden_states.reshape(N, D)
    x16 = x.astype(jnp.bfloat16)
    gwt = gate_weight.T
    wg16 = w_gate.astype(jnp.bfloat16)
    wu16 = w_up.astype(jnp.bfloat16)
    wd16 = w_down.astype(jnp.bfloat16)
    swg16 = sw_gate.astype(jnp.bfloat16)
    swu16 = sw_up.astype(jnp.bfloat16)
    swd16 = sw_down.astype(jnp.bfloat16)

    CT = 256
    idx2d, w2d, ysh = pl.pallas_call(
        _router_body,
        grid=(N // CT,),
        in_specs=[
            pl.BlockSpec((CT, D), lambda t: (t, 0)),
            pl.BlockSpec((D, E), lambda t: (0, 0)),
            pl.BlockSpec((D, FS), lambda t: (0, 0)),
            pl.BlockSpec((D, FS), lambda t: (0, 0)),
            pl.BlockSpec((FS, D), lambda t: (0, 0)),
        ],
        out_specs=[
            pl.BlockSpec((CT, E), lambda t: (t, 0)),
            pl.BlockSpec((CT, E), lambda t: (t, 0)),
            pl.BlockSpec((CT, D), lambda t: (t, 0)),
        ],
        out_shape=[
            jax.ShapeDtypeStruct((N, E), jnp.int32),
            jax.ShapeDtypeStruct((N, E), jnp.float32),
            jax.ShapeDtypeStruct((N, D), jnp.float32),
        ],
    )(x, gwt, swg16, swu16, swd16)

    sorted_tok, pairdst, slots = pl.kernel(
        _meta_body,
        out_type=[
            jax.ShapeDtypeStruct((NROWS,), jnp.int32),
            jax.ShapeDtypeStruct((NROWS,), jnp.int32),
            jax.ShapeDtypeStruct((48,), jnp.int32),
        ],
        mesh=plsc.VectorSubcoreMesh(
            core_axis_name="c", subcore_axis_name="s", num_cores=1),
        compiler_params=pltpu.CompilerParams(needs_layout_passes=False),
        scratch_types=[
            pltpu.VMEM((N,), jnp.int32),
            pltpu.VMEM((N,), jnp.int32),
            pltpu.VMEM((SEG_SZ,), jnp.int32),
            pltpu.VMEM((SEG_SZ,), jnp.int32),
            pltpu.VMEM((128,), jnp.int32),
            pltpu.VMEM((48,), jnp.int32),
        ],
    )(idx2d[:, 0], idx2d[:, 1])

    x_sorted = pl.kernel(
        _xgather_body,
        out_type=jax.ShapeDtypeStruct((NROWS, D), jnp.float32),
        mesh=plsc.VectorSubcoreMesh(core_axis_name="c", subcore_axis_name="s"),
        compiler_params=pltpu.CompilerParams(needs_layout_passes=False),
        scratch_types=[
            pltpu.VMEM((160,), jnp.int32),
            pltpu.VMEM((16,), jnp.int32),
            pltpu.VMEM((96, D), jnp.float32),
            pltpu.SemaphoreType.DMA,
            pltpu.SemaphoreType.DMA,
        ],
    )(x, sorted_tok, slots)

    out_buf = pl.pallas_call(
        _expert_body,
        grid_spec=pltpu.PrefetchScalarGridSpec(
            num_scalar_prefetch=1,
            grid=(NSLOT,),
            in_specs=[
                pl.BlockSpec((T, D), lambda t, m: (t, 0)),
                pl.BlockSpec((1, D, F), lambda t, m: (m[t], 0, 0)),
                pl.BlockSpec((1, D, F), lambda t, m: (m[t], 0, 0)),
                pl.BlockSpec((1, F, D), lambda t, m: (m[t], 0, 0)),
            ],
            out_specs=pl.BlockSpec((T, D), lambda t, m: (t, 0)),
        ),
        out_shape=jax.ShapeDtypeStruct((NROWS, D), jnp.float32),
    )(slots, x_sorted, wg16, wu16, wd16)

    ypairs = pl.kernel(
        _oscatter_body,
        out_type=jax.ShapeDtypeStruct((NPAIR_PAD, D), jnp.float32),
        mesh=plsc.VectorSubcoreMesh(core_axis_name="c", subcore_axis_name="s"),
        compiler_params=pltpu.CompilerParams(needs_layout_passes=False),
        scratch_types=[
            pltpu.VMEM((160,), jnp.int32),
            pltpu.VMEM((16,), jnp.int32),
            pltpu.VMEM((96, D), jnp.float32),
            pltpu.SemaphoreType.DMA,
            pltpu.SemaphoreType.DMA,
        ],
    )(out_buf, pairdst, slots)

    y = pl.pallas_call(
        _combine_body,
        grid=(N // CT,),
        in_specs=[
            pl.BlockSpec((CT, D), lambda t: (t, 0)),
            pl.BlockSpec((CT, D), lambda t: (t, 0)),
            pl.BlockSpec((CT, D), lambda t: (t + N // CT, 0)),
            pl.BlockSpec((CT, E), lambda t: (t, 0)),
        ],
        out_specs=pl.BlockSpec((CT, D), lambda t: (t, 0)),
        out_shape=jax.ShapeDtypeStruct((N, D), jnp.float32),
    )(ysh, ypairs, ypairs, w2d)

    return y.reshape(Bsz, S, D)
